# baseline (speedup 1.0000x reference)
import functools

import jax
import jax.numpy as jnp
from jax import lax
from jax.experimental import pallas as pl
from jax.experimental.pallas import tpu as pltpu
from jax.experimental.pallas import tpu_sc as plsc

NUM_CORES = 2
NUM_SUBCORES = 16
NUM_WORKERS = NUM_CORES * NUM_SUBCORES


@functools.lru_cache(maxsize=None)
def _make_gather(B, D, sub):
    b_per_w = B // NUM_WORKERS
    n_groups = b_per_w // 16
    mesh = plsc.VectorSubcoreMesh(core_axis_name="c", subcore_axis_name="s")

    @functools.partial(
        pl.kernel,
        mesh=mesh,
        out_type=jax.ShapeDtypeStruct((D, B), jnp.float32),
        scratch_types=[
            pltpu.VMEM((b_per_w,), jnp.int32),
            pltpu.VMEM((b_per_w, D), jnp.float32),
            pltpu.VMEM((D, b_per_w), jnp.float32),
            pltpu.HBM((b_per_w, D), jnp.float32),
            pltpu.SemaphoreType.DMA,
        ],
        compiler_params=pltpu.CompilerParams(
            use_tc_tiling_on_sc=True, needs_layout_passes=False
        ),
    )
    def gather_kernel(tab_hbm, idx_hbm, out_hbm, idx_v, gbuf, stage, dummy, sem):
        wid = lax.axis_index("s") * NUM_CORES + lax.axis_index("c")
        base = wid * b_per_w
        pltpu.sync_copy(idx_hbm.at[wid], idx_v)

        def issue_body(g, carry):
            vec = idx_v[pl.ds(g * 16, 16)]
            t_vec = lax.shift_right_logical(vec, 3)
            s_vec = jnp.bitwise_and(vec, sub - 1)
            for l in range(16):
                pltpu.async_copy(
                    tab_hbm.at[t_vec[l], s_vec[l]],
                    gbuf.at[g * 16 + l],
                    sem,
                )
            return carry

        lax.fori_loop(0, n_groups, issue_body, 0)
        pltpu.make_async_copy(dummy, gbuf, sem).wait()

        lane = lax.iota(jnp.int32, 16)

        def transpose_body(g, carry):
            p_vec = lane + g * 16
            for c in range(D):
                col = jnp.full((16,), c, jnp.int32)
                val = plsc.load_gather(gbuf, [p_vec, col])
                plsc.store_scatter(stage, [col, p_vec], val)
            return carry

        lax.fori_loop(0, n_groups, transpose_body, 0)
        pltpu.sync_copy(stage, out_hbm.at[:, pl.ds(base, b_per_w)])

    return gather_kernel


def kernel(x, embed_weight):
    (B,) = x.shape
    V, D = embed_weight.shape
    sub = 8
    tab3 = embed_weight.reshape(V // sub, sub, D)
    idx = x.astype(jnp.int32).reshape(NUM_WORKERS, B // NUM_WORKERS)
    out_t = _make_gather(B, D, sub)(tab3, idx)
    return out_t.T[None]
